# baseline (speedup 1.0000x reference)
import functools

import jax
import jax.numpy as jnp
from jax import lax
from jax.experimental import pallas as pl
from jax.experimental.pallas import tpu as pltpu
from jax.experimental.pallas import tpu_sc as plsc

B = 8
CIN = 3
HW = 512
C = 512
P = 16
K = 1024
D = 256
S = 1024
F = 768
NJ = 4
N_ROWS = B * C * NJ
LOSS_SCALE = 1.0 / float(B * S * C // 2)
NREC = B * CIN * HW * (HW // P)


def _enc_vq_body(xp_ref, we_ref, be_ref, cbt_ref, idx_ref, loss_ref):
    b = pl.program_id(0)
    j = pl.program_id(1)
    xr = xp_ref[0].reshape(CIN, 8, P, 32, P)
    xj = xr.transpose(0, 2, 4, 1, 3).reshape(F, D)
    zj = jnp.dot(we_ref[...], xj, preferred_element_type=jnp.float32)
    zj = zj + be_ref[...]
    dots = jnp.dot(zj, cbt_ref[...], preferred_element_type=jnp.float32)
    x2 = jnp.sum(zj * zj, axis=1, keepdims=True)
    e2 = jnp.sum(cbt_ref[...] * cbt_ref[...], axis=0, keepdims=True)
    dist = x2 - 2.0 * dots + e2
    mval = jnp.min(dist, axis=1, keepdims=True)
    iota = lax.broadcasted_iota(jnp.int32, dist.shape, 1)
    idxj = jnp.min(jnp.where(dist <= mval, iota, jnp.int32(2**30)),
                   axis=1, keepdims=True)
    idx_ref[0] = idxj

    @pl.when((b == 0) & (j == 0))
    def _init():
        loss_ref[0, 0] = 0.0

    loss_ref[0, 0] += jnp.sum(mval) * LOSS_SCALE


def _dec_body(zq_ref, wd_ref, bd_ref, out_ref):
    m = lax.dot_general(zq_ref[0, 0], wd_ref[...],
                        (((0,), (1,)), ((), ())),
                        preferred_element_type=jnp.float32)
    out_ref[0] = m + bd_ref[...]


def _sc_pipelined_gather(tab_hbm, idx_hbm, out_hbm, idx_v, buf0, buf1,
                         sem0, sem1, *, base, n_rec, chunk):
    pltpu.sync_copy(idx_hbm.at[pl.ds(base, n_rec)], idx_v)
    bufs = (buf0, buf1)
    sems = (sem0, sem1)
    nch = n_rec // chunk
    cps = {}
    for t in range(min(2, nch)):
        cps[t] = pltpu.async_copy(
            tab_hbm.at[idx_v.at[pl.ds(t * chunk, chunk)]], bufs[t], sems[t])
    for t in range(nch):
        cps[t].wait()
        pltpu.sync_copy(bufs[t % 2], out_hbm.at[pl.ds(base + t * chunk, chunk)])
        if t + 2 < nch:
            cps[t + 2] = pltpu.async_copy(
                tab_hbm.at[idx_v.at[pl.ds((t + 2) * chunk, chunk)]],
                bufs[t % 2], sems[t % 2])


@functools.cache
def _make_sc_gather():
    info = plsc.get_sparse_core_info()
    nw = info.num_cores * info.num_subcores
    rows_per_w = N_ROWS // nw
    chunk = 128

    @functools.partial(
        pl.kernel,
        out_type=jax.ShapeDtypeStruct((N_ROWS, D), jnp.float32),
        mesh=plsc.VectorSubcoreMesh(core_axis_name="c", subcore_axis_name="s"),
        scratch_types=[
            pltpu.VMEM((rows_per_w,), jnp.int32),
            pltpu.VMEM((chunk, D), jnp.float32),
            pltpu.VMEM((chunk, D), jnp.float32),
            pltpu.SemaphoreType.DMA,
            pltpu.SemaphoreType.DMA,
        ],
    )
    def _sc_gather(cb_hbm, idx_hbm, out_hbm, idx_v, buf0, buf1, sem0, sem1):
        wid = lax.axis_index("s") * info.num_cores + lax.axis_index("c")
        _sc_pipelined_gather(cb_hbm, idx_hbm, out_hbm, idx_v, buf0, buf1,
                             sem0, sem1, base=wid * rows_per_w,
                             n_rec=rows_per_w, chunk=chunk)

    return _sc_gather


@functools.cache
def _make_sc_depatch():
    info = plsc.get_sparse_core_info()
    nw = info.num_cores * info.num_subcores
    recs_per_w = NREC // nw
    chunk = 2048

    @functools.partial(
        pl.kernel,
        out_type=jax.ShapeDtypeStruct((NREC, P), jnp.float32),
        mesh=plsc.VectorSubcoreMesh(core_axis_name="c", subcore_axis_name="s"),
        scratch_types=[
            pltpu.VMEM((recs_per_w,), jnp.int32),
            pltpu.VMEM((chunk, P), jnp.float32),
            pltpu.VMEM((chunk, P), jnp.float32),
            pltpu.SemaphoreType.DMA,
            pltpu.SemaphoreType.DMA,
        ],
    )
    def _sc_depatch(tab_hbm, idx_hbm, out_hbm, idx_v, buf0, buf1, sem0, sem1):
        wid = lax.axis_index("s") * info.num_cores + lax.axis_index("c")
        _sc_pipelined_gather(tab_hbm, idx_hbm, out_hbm, idx_v, buf0, buf1,
                             sem0, sem1, base=wid * recs_per_w,
                             n_rec=recs_per_w, chunk=chunk)

    return _sc_depatch


def _depatch_indices():
    n = jnp.arange(NREC, dtype=jnp.int32)
    w = n % 32
    y = (n // 32) % HW
    o = (n // (32 * HW)) % CIN
    b = n // (32 * HW * CIN)
    h = y // P
    r = y % P
    return (b * S + h * 32 + w) * (F // P) + r * CIN + o


def kernel(X, W_enc, b_enc, codebook, W_dec, b_dec):
    We = W_enc.reshape(C, F)
    cbT = codebook.T
    be = b_enc[:, None]
    Wd = W_dec[::-1, ::-1].transpose(0, 3, 1, 2).reshape(F, C)
    bd = jnp.tile(jnp.repeat(b_dec, P), P)[None, :]

    idx, loss = pl.pallas_call(
        _enc_vq_body,
        grid=(B, NJ),
        in_specs=[
            pl.BlockSpec((1, CIN, 128, HW), lambda b, j: (b, 0, j, 0)),
            pl.BlockSpec((C, F), lambda b, j: (0, 0)),
            pl.BlockSpec((C, 1), lambda b, j: (0, 0)),
            pl.BlockSpec((D, K), lambda b, j: (0, 0)),
        ],
        out_specs=[
            pl.BlockSpec((1, C, 1), lambda b, j: (b * NJ + j, 0, 0)),
            pl.BlockSpec(memory_space=pltpu.SMEM, block_shape=(1, 1),
                         index_map=lambda b, j: (0, 0)),
        ],
        out_shape=[
            jax.ShapeDtypeStruct((B * NJ, C, 1), jnp.int32),
            jax.ShapeDtypeStruct((1, 1), jnp.float32),
        ],
    )(X, We, be, cbT)

    zq = _make_sc_gather()(codebook, idx.reshape(N_ROWS))
    zq = zq.reshape(B, NJ, C, D)

    out_sf = pl.pallas_call(
        _dec_body,
        grid=(B, NJ),
        in_specs=[
            pl.BlockSpec((1, 1, C, D), lambda b, j: (b, j, 0, 0)),
            pl.BlockSpec((F, C), lambda b, j: (0, 0)),
            pl.BlockSpec((1, F), lambda b, j: (0, 0)),
        ],
        out_specs=pl.BlockSpec((1, D, F), lambda b, j: (b, j, 0)),
        out_shape=jax.ShapeDtypeStruct((B, S, F), jnp.float32),
    )(zq, Wd, bd)

    out_rec = jnp.take(out_sf.reshape(NREC, P), _depatch_indices(), axis=0)
    out = out_rec.reshape(B, CIN, HW, HW)
    return out, loss[0, 0]


# speedup vs baseline: 2.2899x; 2.2899x over previous
import functools

import jax
import jax.numpy as jnp
from jax import lax
from jax.experimental import pallas as pl
from jax.experimental.pallas import tpu as pltpu
from jax.experimental.pallas import tpu_sc as plsc

B = 8
CIN = 3
HW = 512
C = 512
P = 16
K = 1024
D = 256
S = 1024
F = 768
NJ = 4
N_ROWS = B * C * NJ
LOSS_SCALE = 1.0 / float(B * S * C // 2)
NREC = B * CIN * HW * (HW // P)


def _enc_vq_body(xp_ref, we_ref, be_ref, cbt_ref, idx_ref, loss_ref):
    b = pl.program_id(0)
    j = pl.program_id(1)
    xr = xp_ref[0].reshape(CIN, 8, P, 32, P)
    xj = xr.transpose(0, 2, 4, 1, 3).reshape(F, D)
    zj = jnp.dot(we_ref[...], xj, preferred_element_type=jnp.float32)
    zj = zj + be_ref[...]
    dots = jnp.dot(zj, cbt_ref[...], preferred_element_type=jnp.float32)
    x2 = jnp.sum(zj * zj, axis=1, keepdims=True)
    e2 = jnp.sum(cbt_ref[...] * cbt_ref[...], axis=0, keepdims=True)
    dist = x2 - 2.0 * dots + e2
    mval = jnp.min(dist, axis=1, keepdims=True)
    iota = lax.broadcasted_iota(jnp.int32, dist.shape, 1)
    idxj = jnp.min(jnp.where(dist <= mval, iota, jnp.int32(2**30)),
                   axis=1, keepdims=True)
    idx_ref[0] = idxj

    @pl.when((b == 0) & (j == 0))
    def _init():
        loss_ref[0, 0] = 0.0

    loss_ref[0, 0] += jnp.sum(mval) * LOSS_SCALE


def _dec_body(zq_ref, wd_ref, bd_ref, out_ref):
    m = lax.dot_general(zq_ref[0, 0], wd_ref[...],
                        (((0,), (1,)), ((), ())),
                        preferred_element_type=jnp.float32)
    out_ref[0] = m + bd_ref[...]


def _sc_pipelined_gather(tab_hbm, idx_hbm, out_hbm, idx_v, buf0, buf1,
                         sem0, sem1, *, base, n_rec, chunk):
    pltpu.sync_copy(idx_hbm.at[pl.ds(base, n_rec)], idx_v)
    bufs = (buf0, buf1)
    sems = (sem0, sem1)
    nch = n_rec // chunk
    cps = {}
    for t in range(min(2, nch)):
        cps[t] = pltpu.async_copy(
            tab_hbm.at[idx_v.at[pl.ds(t * chunk, chunk)]], bufs[t], sems[t])
    for t in range(nch):
        cps[t].wait()
        pltpu.sync_copy(bufs[t % 2], out_hbm.at[pl.ds(base + t * chunk, chunk)])
        if t + 2 < nch:
            cps[t + 2] = pltpu.async_copy(
                tab_hbm.at[idx_v.at[pl.ds((t + 2) * chunk, chunk)]],
                bufs[t % 2], sems[t % 2])


@functools.cache
def _make_sc_gather():
    info = plsc.get_sparse_core_info()
    nw = info.num_cores * info.num_subcores
    rows_per_w = N_ROWS // nw
    chunk = 128

    @functools.partial(
        pl.kernel,
        out_type=jax.ShapeDtypeStruct((N_ROWS, D), jnp.float32),
        mesh=plsc.VectorSubcoreMesh(core_axis_name="c", subcore_axis_name="s"),
        scratch_types=[
            pltpu.VMEM((rows_per_w,), jnp.int32),
            pltpu.VMEM((chunk, D), jnp.float32),
            pltpu.VMEM((chunk, D), jnp.float32),
            pltpu.SemaphoreType.DMA,
            pltpu.SemaphoreType.DMA,
        ],
    )
    def _sc_gather(cb_hbm, idx_hbm, out_hbm, idx_v, buf0, buf1, sem0, sem1):
        wid = lax.axis_index("s") * info.num_cores + lax.axis_index("c")
        _sc_pipelined_gather(cb_hbm, idx_hbm, out_hbm, idx_v, buf0, buf1,
                             sem0, sem1, base=wid * rows_per_w,
                             n_rec=rows_per_w, chunk=chunk)

    return _sc_gather


@functools.cache
def _make_sc_depatch():
    info = plsc.get_sparse_core_info()
    nw = info.num_cores * info.num_subcores
    n_slabs = B * 32
    slabs_per_w = n_slabs // nw

    slab_sz = 32 * F
    dst_sz = CIN * P * HW
    o_sz = P * HW

    @functools.partial(
        pl.kernel,
        out_type=jax.ShapeDtypeStruct((B * CIN * HW * HW,), jnp.float32),
        mesh=plsc.VectorSubcoreMesh(core_axis_name="c", subcore_axis_name="s"),
        scratch_types=[
            pltpu.VMEM((slab_sz,), jnp.float32),
            pltpu.VMEM((slab_sz,), jnp.float32),
            pltpu.VMEM((dst_sz,), jnp.float32),
            pltpu.SemaphoreType.DMA,
            pltpu.SemaphoreType.DMA,
        ],
    )
    def _sc_depatch(sf_hbm, out_hbm, slab0, slab1, dst, sem0, sem1):
        wid = lax.axis_index("s") * info.num_cores + lax.axis_index("c")
        base = wid * slabs_per_w
        slabs = (slab0, slab1)
        sems = (sem0, sem1)
        cps = {}
        for t in range(2):
            cps[t] = pltpu.async_copy(
                sf_hbm.at[pl.ds((base + t) * slab_sz, slab_sz)],
                slabs[t], sems[t])
        for t in range(slabs_per_w):
            cps[t].wait()
            slab = slabs[t % 2]

            def _reorder(k, _, slab=slab):
                o = k // P
                r = k % P
                col = r * (CIN * P) + o * P
                for w in range(32):
                    dst[pl.ds(k * HW + w * P, P)] = slab[pl.ds(w * F + col, P)]
                return 0

            lax.fori_loop(0, CIN * P, _reorder, 0)
            sid = base + t
            b = sid // 32
            h = sid % 32
            for o in range(CIN):
                dst0 = ((b * CIN + o) * HW + h * P) * HW
                pltpu.sync_copy(dst.at[pl.ds(o * o_sz, o_sz)],
                                out_hbm.at[pl.ds(dst0, o_sz)])
            if t + 2 < slabs_per_w:
                cps[t + 2] = pltpu.async_copy(
                    sf_hbm.at[pl.ds((base + t + 2) * slab_sz, slab_sz)],
                    slabs[t % 2], sems[t % 2])

    return _sc_depatch


def kernel(X, W_enc, b_enc, codebook, W_dec, b_dec):
    We = W_enc.reshape(C, F)
    cbT = codebook.T
    be = b_enc[:, None]
    Wd = W_dec[::-1, ::-1].transpose(0, 3, 1, 2).reshape(F, C)
    bd = jnp.tile(jnp.repeat(b_dec, P), P)[None, :]

    idx, loss = pl.pallas_call(
        _enc_vq_body,
        grid=(B, NJ),
        in_specs=[
            pl.BlockSpec((1, CIN, 128, HW), lambda b, j: (b, 0, j, 0)),
            pl.BlockSpec((C, F), lambda b, j: (0, 0)),
            pl.BlockSpec((C, 1), lambda b, j: (0, 0)),
            pl.BlockSpec((D, K), lambda b, j: (0, 0)),
        ],
        out_specs=[
            pl.BlockSpec((1, C, 1), lambda b, j: (b * NJ + j, 0, 0)),
            pl.BlockSpec(memory_space=pltpu.SMEM, block_shape=(1, 1),
                         index_map=lambda b, j: (0, 0)),
        ],
        out_shape=[
            jax.ShapeDtypeStruct((B * NJ, C, 1), jnp.int32),
            jax.ShapeDtypeStruct((1, 1), jnp.float32),
        ],
    )(X, We, be, cbT)

    zq = _make_sc_gather()(codebook, idx.reshape(N_ROWS))
    zq = zq.reshape(B, NJ, C, D)

    out_sf = pl.pallas_call(
        _dec_body,
        grid=(B, NJ),
        in_specs=[
            pl.BlockSpec((1, 1, C, D), lambda b, j: (b, j, 0, 0)),
            pl.BlockSpec((F, C), lambda b, j: (0, 0)),
            pl.BlockSpec((1, F), lambda b, j: (0, 0)),
        ],
        out_specs=pl.BlockSpec((1, D, F), lambda b, j: (b, j, 0)),
        out_shape=jax.ShapeDtypeStruct((B, S, F), jnp.float32),
    )(zq, Wd, bd)

    out_rows = _make_sc_depatch()(out_sf.reshape(B * S * F))
    out = out_rows.reshape(B, CIN, HW, HW)
    return out, loss[0, 0]
